# Initial kernel scaffold; baseline (speedup 1.0000x reference)
import jax
import jax.numpy as jnp
from jax.experimental import pallas as pl


def kernel(x, edge_index, W_lift, b_lift, W1, b1, W2, b2, Wr, br):
    raise NotImplementedError("write your pallas kernel here")


# speedup vs baseline: 9.1686x; 9.1686x over previous
import jax
import jax.numpy as jnp
from jax.experimental import pallas as pl

_N = 22
_E = 484


def _body(src_ref, dst_ref, x_ref, wl_ref, bl_ref, w1_ref, b1_ref,
          w2_ref, b2_ref, wr_ref, br_ref, out_ref):
    f32 = jnp.float32
    nodes = jax.lax.broadcasted_iota(jnp.int32, (_N, _E), 0)
    d_oh = (dst_ref[...] == nodes).astype(f32)
    s_oh = (src_ref[...] == nodes).astype(f32)
    adj = jax.lax.dot_general(d_oh, s_oh, (((1,), (1,)), ((), ())),
                              preferred_element_type=f32)
    h = jnp.maximum(
        jnp.dot(x_ref[...], wl_ref[...], preferred_element_type=f32)
        + bl_ref[...], 0.0)
    agg = jnp.dot(adj, h, preferred_element_type=f32)
    h = jnp.maximum(
        jnp.dot(agg, w1_ref[...], preferred_element_type=f32)
        + b1_ref[...], 0.0)
    agg = jnp.dot(adj, h, preferred_element_type=f32)
    h = jnp.maximum(
        jnp.dot(agg, w2_ref[...], preferred_element_type=f32)
        + b2_ref[...], 0.0)
    out_ref[...] = jnp.sum(h * wr_ref[...])[None, None] + br_ref[...]


def kernel(x, edge_index, W_lift, b_lift, W1, b1, W2, b2, Wr, br):
    src = edge_index[0].reshape(1, _E)
    dst = edge_index[1].reshape(1, _E)
    out = pl.pallas_call(
        _body,
        out_shape=jax.ShapeDtypeStruct((1, 1), jnp.float32),
    )(src, dst, x, W_lift, b_lift.reshape(1, -1), W1, b1.reshape(1, -1),
      W2, b2.reshape(1, -1), Wr.reshape(_N, 5), br.reshape(1, 1))
    return out
